# baseline (speedup 1.0000x reference)
import functools

import numpy as np

import jax
import jax.numpy as jnp
from jax import lax
from jax.experimental import pallas as pl
from jax.experimental.pallas import tpu as pltpu
from jax.experimental.pallas import tpu_sc as plsc

N = 10000
E = 160000
H = 256
HH = H // 2
RBF = 16

NC = 2
NS = 16
K = 40
NCHT = 250
NCHM = 248
EPT = NCHT * K
OUT_ROWS = (N // NS // 8) * 8


def _gate_body(rbf_ref, W1_ref, b1_ref, W2_ref, b2_ref, out_ref):
    t = jnp.dot(rbf_ref[...], W1_ref[...] * 0.5,
                preferred_element_type=jnp.float32)
    t = t + b1_ref[...] * 0.5
    g = t * (jnp.tanh(t) + 1.0)
    z = jnp.dot(g.astype(jnp.bfloat16), W2_ref[...] * 0.5,
                preferred_element_type=jnp.float32) + b2_ref[...] * 0.5
    v = jnp.tanh(z)
    out_ref[0] = v[:, :HH]
    out_ref[1] = v[:, HH:]


_BE = 2000


def _gate_call(rbf, W1, b1, W2, b2):
    return pl.pallas_call(
        _gate_body,
        grid=(E // _BE,),
        in_specs=[
            pl.BlockSpec((_BE, RBF), lambda i: (i, 0)),
            pl.BlockSpec((RBF, H), lambda i: (0, 0)),
            pl.BlockSpec((1, H), lambda i: (0, 0)),
            pl.BlockSpec((H, H), lambda i: (0, 0)),
            pl.BlockSpec((1, H), lambda i: (0, 0)),
        ],
        out_specs=pl.BlockSpec((2, _BE, HH), lambda i: (0, i, 0)),
        out_shape=jax.ShapeDtypeStruct((2, E, HH), jnp.float32),
    )(rbf, W1, b1, W2, b2)


@functools.cache
def _sc_aggr_built():
    mesh = plsc.VectorSubcoreMesh(
        core_axis_name="c", subcore_axis_name="s", num_cores=NC, num_subcores=NS
    )
    return functools.partial(
        pl.kernel,
        out_type=jax.ShapeDtypeStruct((2, N, HH), jnp.float32),
        mesh=mesh,
        scratch_types=(
            [pltpu.VMEM((EPT,), jnp.int32)]
            + [pltpu.VMEM((K,), jnp.int32) for _ in range(4)]
            + [pltpu.VMEM((K, HH), jnp.float32) for _ in range(2)]
            + [pltpu.VMEM((K, HH), jnp.float32) for _ in range(2)]
            + [pltpu.VMEM((K, HH), jnp.float32) for _ in range(2)]
            + [pltpu.VMEM_SHARED((N, HH), jnp.float32)]
            + [pltpu.SemaphoreType.DMA for _ in range(10)]
        ),
    )(_sc_aggr_body)


def _sc_call(h2, src, dst, gcat):
    return _sc_aggr_built()(h2, src, dst, gcat)


def _sc_aggr_body(h2, srcx, dstx, gcat, out,
                  src_i, di0, di1, di2, di3,
                  hv0, hv1, gv0, gv1, mv0, mv1, acc,
                  sx0, sx1, sx2, sx3, sh0, sh1, sg0, sg1, ss0, ss1):
    c = lax.axis_index("c")
    s = lax.axis_index("s")
    di = (di0, di1, di2, di3)
    sx = (sx0, sx1, sx2, sx3)
    hv = (hv0, hv1)
    gv = (gv0, gv1)
    mv = (mv0, mv1)
    sh = (sh0, sh1)
    sg = (sg0, sg1)
    ss = (ss0, ss1)

    ebase = s * EPT

    def issue_dst_idx(j, x):
        e0 = ebase + j * K
        pltpu.make_async_copy(dstx.at[pl.ds(e0, K)], di[x], sx[x]).start()

    def wait_dst_idx(j, x):
        e0 = ebase + j * K
        pltpu.make_async_copy(dstx.at[pl.ds(e0, K)], di[x], sx[x]).wait()

    def issue_loads(j, t):
        pltpu.make_async_copy(
            h2.at[src_i.at[pl.ds(j * K, K)]], hv[t], sh[t]).start()
        pltpu.make_async_copy(
            gcat.at[c, pl.ds(ebase + j * K, K)], gv[t], sg[t]).start()

    def wait_loads(j, t):
        pltpu.make_async_copy(
            h2.at[src_i.at[pl.ds(j * K, K)]], hv[t], sh[t]).wait()
        pltpu.make_async_copy(
            gcat.at[c, pl.ds(ebase + j * K, K)], gv[t], sg[t]).wait()

    def issue_scatter(t, x):
        pltpu.async_copy(mv[t], acc.at[di[x]], ss[t], add=True)

    def wait_scatter(t, x):
        pltpu.make_async_copy(mv[t], acc.at[di[x]], ss[t]).wait()

    def multiply(t):
        @plsc.parallel_loop(0, K, unroll=2)
        def mrow(r):
            for q in range(HH // 16):
                sl = pl.ds(q * 16, 16)
                mv[t][r, sl] = (gv[t][r, sl] + 1.0) * hv[t][r, sl]

    pltpu.sync_copy(srcx.at[pl.ds(ebase, EPT)], src_i)

    def xform(i, carry):
        sl = pl.ds(i * 16, 16)
        src_i[sl] = src_i[sl] * 2 + c
        return carry

    lax.fori_loop(0, EPT // 16, xform, 0)
    issue_dst_idx(0, 0)
    issue_dst_idx(1, 1)
    issue_loads(0, 0)

    def zrow(r2, carry):
        zvec = jnp.zeros((16,), jnp.float32)
        for dr in range(2):
            r = r2 * 2 + dr
            for q in range(HH // 16):
                mv0[r, pl.ds(q * 16, 16)] = zvec
        return carry

    lax.fori_loop(0, K // 2, zrow, 0)
    zbase = s * (N // NS)
    npiece = (N // NS) // K
    for piece in range(npiece):
        pltpu.sync_copy(mv0, acc.at[pl.ds(zbase + piece * K, K)])
    rem = N // NS - npiece * K
    pltpu.sync_copy(mv0.at[pl.ds(0, rem)],
                    acc.at[pl.ds(zbase + npiece * K, rem)])
    plsc.subcore_barrier()

    NQ = NCHM // 4

    def quad(q, carry):
        for u in range(4):
            j = q * 4 + u
            t = u % 2
            if u < 3:
                issue_loads(j + 1, 1 - t)
            else:
                @pl.when(q < NQ - 1)
                def _next_quad_loads():
                    issue_loads(j + 1, 1 - t)

            wait_loads(j, t)

            if u < 2:
                @pl.when(q >= 1)
                def _wait_prev_scatter():
                    wait_scatter(t, u + 2)

                issue_dst_idx(j + 2, u + 2)
            else:
                wait_scatter(t, u - 2)

                @pl.when(q < NQ - 1)
                def _refill_dst():
                    issue_dst_idx(j + 2, u - 2)

            multiply(t)
            wait_dst_idx(j, u)
            issue_scatter(t, u)

        return carry

    lax.fori_loop(0, NQ, quad, 0)
    wait_scatter(0, 2)
    wait_scatter(1, 3)

    for jj, tt in ((NCHM, 0), (NCHM + 1, 1)):
        issue_loads(jj, tt)
        pltpu.sync_copy(dstx.at[pl.ds(ebase + jj * K, K)], di[tt])
        wait_loads(jj, tt)
        multiply(tt)
        pltpu.sync_copy(mv[tt], acc.at[di[tt]], add=True)

    plsc.subcore_barrier()

    r0 = s * OUT_ROWS
    pltpu.sync_copy(acc.at[pl.ds(r0, OUT_ROWS)], out.at[c, pl.ds(r0, OUT_ROWS)])

    @pl.when(s == NS - 1)
    def _tail():
        t0 = NS * OUT_ROWS
        pltpu.sync_copy(acc.at[pl.ds(t0, N - NS * OUT_ROWS)],
                        out.at[c, pl.ds(t0, N - NS * OUT_ROWS)])


def _upd_body(h_ref, a_ref, U1h_ref, U1a_ref, c1_ref, U2_ref, c2_ref, out_ref):
    hblk = h_ref[...]
    t = jnp.dot(hblk.astype(jnp.bfloat16), U1h_ref[...],
                preferred_element_type=jnp.float32)
    t = t + jnp.dot(a_ref[0].astype(jnp.bfloat16), U1a_ref[0] * 0.5,
                    preferred_element_type=jnp.float32)
    t = t + jnp.dot(a_ref[1].astype(jnp.bfloat16), U1a_ref[1] * 0.5,
                    preferred_element_type=jnp.float32)
    t = t + c1_ref[...]
    th = 0.5 * t
    u = th * (jnp.tanh(th) + 1.0)
    out_ref[...] = hblk + jnp.dot(u.astype(jnp.bfloat16), U2_ref[...],
                                  preferred_element_type=jnp.float32) + c2_ref[...]


_BN = 1000


def _upd_call(h, aggr_cat, U1h, U1a, c1, U2, c2):
    return pl.pallas_call(
        _upd_body,
        grid=(N // _BN,),
        in_specs=[
            pl.BlockSpec((_BN, H), lambda i: (i, 0)),
            pl.BlockSpec((2, _BN, HH), lambda i: (0, i, 0)),
            pl.BlockSpec((H, H), lambda i: (0, 0)),
            pl.BlockSpec((2, HH, H), lambda i: (0, 0, 0)),
            pl.BlockSpec((1, H), lambda i: (0, 0)),
            pl.BlockSpec((H, H), lambda i: (0, 0)),
            pl.BlockSpec((1, H), lambda i: (0, 0)),
        ],
        out_specs=pl.BlockSpec((_BN, H), lambda i: (i, 0)),
        out_shape=jax.ShapeDtypeStruct((N, H), jnp.float32),
    )(h, aggr_cat, U1h, U1a, c1, U2, c2)


_PERM = np.concatenate([
    np.stack([np.arange(64) + 128 * hh, np.arange(64) + 128 * hh + 64],
             axis=1).reshape(-1)
    for hh in (0, 1)
])


def kernel(h, edge_index, rbf, W1, b1, W2, b2, U1, c1, U2, c2):
    src = edge_index[0]
    dst = edge_index[1]
    gate_cat = _gate_call(rbf, W1, b1.reshape(1, H),
                          W2.astype(jnp.bfloat16), b2.reshape(1, H))
    h2 = h.reshape(2 * N, HH)
    aggr_cat = _sc_call(h2, src, dst, gate_cat)
    return _upd_call(
        h,
        aggr_cat,
        U1[:H].astype(jnp.bfloat16),
        U1[H:].reshape(2, HH, H).astype(jnp.bfloat16),
        c1.reshape(1, H),
        U2.astype(jnp.bfloat16),
        c2.reshape(1, H),
    )


# speedup vs baseline: 1.1098x; 1.1098x over previous
import functools

import numpy as np

import jax
import jax.numpy as jnp
from jax import lax
from jax.experimental import pallas as pl
from jax.experimental.pallas import tpu as pltpu
from jax.experimental.pallas import tpu_sc as plsc

N = 10000
E = 160000
H = 256
HH = H // 2
RBF = 16

NC = 2
NS = 16
K = 64
NCHT = 156
NPAIR = NCHT // 2
EPT = NCHT * K
OUT_ROWS = (N // NS // 8) * 8


def _gate_body(rbf_ref, W1_ref, b1_ref, W2_ref, b2_ref, out_ref):
    t = jnp.dot(rbf_ref[...], W1_ref[...] * 0.5,
                preferred_element_type=jnp.float32)
    t = t + b1_ref[...] * 0.5
    g = t * (jnp.tanh(t) + 1.0)
    z = jnp.dot(g.astype(jnp.bfloat16), W2_ref[...] * 0.5,
                preferred_element_type=jnp.float32) + b2_ref[...] * 0.5
    v = jnp.tanh(z)
    out_ref[0] = v[:, :HH]
    out_ref[1] = v[:, HH:]


_BE = 4000


def _gate_call(rbf, W1, b1, W2, b2):
    return pl.pallas_call(
        _gate_body,
        grid=(E // _BE,),
        in_specs=[
            pl.BlockSpec((_BE, RBF), lambda i: (i, 0)),
            pl.BlockSpec((RBF, H), lambda i: (0, 0)),
            pl.BlockSpec((1, H), lambda i: (0, 0)),
            pl.BlockSpec((H, H), lambda i: (0, 0)),
            pl.BlockSpec((1, H), lambda i: (0, 0)),
        ],
        out_specs=pl.BlockSpec((2, _BE, HH), lambda i: (0, i, 0)),
        out_shape=jax.ShapeDtypeStruct((2, E, HH), jnp.float32),
    )(rbf, W1, b1, W2, b2)


@functools.cache
def _sc_aggr_built():
    mesh = plsc.VectorSubcoreMesh(
        core_axis_name="c", subcore_axis_name="s", num_cores=NC, num_subcores=NS
    )
    return functools.partial(
        pl.kernel,
        out_type=jax.ShapeDtypeStruct((2, N, HH), jnp.float32),
        mesh=mesh,
        scratch_types=(
            [pltpu.VMEM((K,), jnp.int32) for _ in range(4)]
            + [pltpu.VMEM((K,), jnp.int32) for _ in range(4)]
            + [pltpu.VMEM((K, HH), jnp.float32) for _ in range(2)]
            + [pltpu.VMEM((K, HH), jnp.float32) for _ in range(2)]
            + [pltpu.VMEM((K, HH), jnp.float32) for _ in range(2)]
            + [pltpu.VMEM_SHARED((N, HH), jnp.float32)]
            + [pltpu.SemaphoreType.DMA for _ in range(10)]
        ),
    )(_sc_aggr_body)


def _sc_call(h2, src, dst, gcat):
    return _sc_aggr_built()(h2, src, dst, gcat)


def _sc_aggr_body(h2, srcx, dstx, gcat, out,
                  si0, si1, si2, si3, di0, di1, di2, di3,
                  hv0, hv1, gv0, gv1, mv0, mv1, acc,
                  sx0, sx1, sx2, sx3, sh0, sh1, sg0, sg1, ss0, ss1):
    c = lax.axis_index("c")
    s = lax.axis_index("s")
    si = (si0, si1, si2, si3)
    di = (di0, di1, di2, di3)
    sx = (sx0, sx1, sx2, sx3)
    hv = (hv0, hv1)
    gv = (gv0, gv1)
    mv = (mv0, mv1)
    sh = (sh0, sh1)
    sg = (sg0, sg1)
    ss = (ss0, ss1)

    ebase = s * EPT

    def issue_src_idx(j, x):
        e0 = ebase + j * K
        pltpu.make_async_copy(srcx.at[pl.ds(e0, K)], si[x], sx[x]).start()

    def issue_dst_idx(j, x):
        e0 = ebase + j * K
        pltpu.make_async_copy(dstx.at[pl.ds(e0, K)], di[x], sx[x]).start()

    def wait_idx_and_xform(j, x):
        e0 = ebase + j * K
        pltpu.make_async_copy(srcx.at[pl.ds(e0, K)], si[x], sx[x]).wait()
        pltpu.make_async_copy(dstx.at[pl.ds(e0, K)], di[x], sx[x]).wait()
        for q in range(K // 16):
            sl = pl.ds(q * 16, 16)
            si[x][sl] = si[x][sl] * 2 + c

    def issue_loads(j, t, x):
        pltpu.make_async_copy(h2.at[si[x]], hv[t], sh[t]).start()
        pltpu.make_async_copy(
            gcat.at[c, pl.ds(ebase + j * K, K)], gv[t], sg[t]).start()

    def wait_loads(j, t, x):
        pltpu.make_async_copy(h2.at[si[x]], hv[t], sh[t]).wait()
        pltpu.make_async_copy(
            gcat.at[c, pl.ds(ebase + j * K, K)], gv[t], sg[t]).wait()

    def issue_scatter(t, x):
        pltpu.async_copy(mv[t], acc.at[di[x]], ss[t], add=True)

    def wait_scatter(t, x):
        pltpu.make_async_copy(mv[t], acc.at[di[x]], ss[t]).wait()

    def multiply(t):
        @plsc.parallel_loop(0, K, unroll=2)
        def mrow(r):
            for q in range(HH // 16):
                sl = pl.ds(q * 16, 16)
                mv[t][r, sl] = (gv[t][r, sl] + 1.0) * hv[t][r, sl]

    for j in range(4):
        issue_src_idx(j, j)
    issue_dst_idx(0, 0)
    issue_dst_idx(1, 1)
    wait_idx_and_xform(0, 0)
    issue_loads(0, 0, 0)

    def zrow(r2, carry):
        zvec = jnp.zeros((16,), jnp.float32)
        for dr in range(2):
            r = r2 * 2 + dr
            for q in range(HH // 16):
                mv0[r, pl.ds(q * 16, 16)] = zvec
        return carry

    lax.fori_loop(0, K // 2, zrow, 0)
    zbase = s * (N // NS)
    npiece = (N // NS) // K
    for piece in range(npiece):
        pltpu.sync_copy(mv0, acc.at[pl.ds(zbase + piece * K, K)])
    rem = N // NS - npiece * K
    pltpu.sync_copy(mv0.at[pl.ds(0, rem)],
                    acc.at[pl.ds(zbase + npiece * K, rem)])
    plsc.subcore_barrier()

    NQ = NCHT // 4

    def quad(q, carry):
        for u in range(4):
            j = q * 4 + u
            t = u % 2
            if u < 3:
                wait_idx_and_xform(j + 1, u + 1)
                issue_loads(j + 1, 1 - t, u + 1)
            else:
                @pl.when(q < NQ - 1)
                def _next_quad_loads():
                    wait_idx_and_xform(j + 1, 0)
                    issue_loads(j + 1, 1 - t, 0)

            wait_loads(j, t, u)

            if u < 2:
                @pl.when(q >= 1)
                def _wait_prev_scatter():
                    wait_scatter(t, u + 2)

                issue_dst_idx(j + 2, u + 2)
            else:
                wait_scatter(t, u - 2)

                @pl.when(q < NQ - 1)
                def _refill_dst():
                    issue_dst_idx(j + 2, u - 2)

            multiply(t)
            issue_scatter(t, u)

            @pl.when(q < NQ - 1)
            def _refill_src():
                issue_src_idx(j + 4, u)

        return carry

    lax.fori_loop(0, NQ, quad, 0)
    wait_scatter(0, 2)
    wait_scatter(1, 3)

    @pl.when(s < 4)
    def _extra():
        ec = NS * NCHT + s
        e0 = ec * K
        pltpu.sync_copy(srcx.at[pl.ds(e0, K)], si0)
        pltpu.sync_copy(dstx.at[pl.ds(e0, K)], di0)
        for q in range(K // 16):
            sl = pl.ds(q * 16, 16)
            si0[sl] = si0[sl] * 2 + c
        pltpu.async_copy(h2.at[si0], hv0, sh0).wait()
        pltpu.sync_copy(gcat.at[c, pl.ds(e0, K)], gv0)
        multiply(0)
        pltpu.sync_copy(mv0, acc.at[di0], add=True)

    plsc.subcore_barrier()

    r0 = s * OUT_ROWS
    pltpu.sync_copy(acc.at[pl.ds(r0, OUT_ROWS)], out.at[c, pl.ds(r0, OUT_ROWS)])

    @pl.when(s == NS - 1)
    def _tail():
        t0 = NS * OUT_ROWS
        pltpu.sync_copy(acc.at[pl.ds(t0, N - NS * OUT_ROWS)],
                        out.at[c, pl.ds(t0, N - NS * OUT_ROWS)])


def _upd_body(h_ref, a_ref, U1h_ref, U1a_ref, c1_ref, U2_ref, c2_ref, out_ref):
    hblk = h_ref[...]
    t = jnp.dot(hblk.astype(jnp.bfloat16), U1h_ref[...],
                preferred_element_type=jnp.float32)
    t = t + jnp.dot(a_ref[0].astype(jnp.bfloat16), U1a_ref[0] * 0.5,
                    preferred_element_type=jnp.float32)
    t = t + jnp.dot(a_ref[1].astype(jnp.bfloat16), U1a_ref[1] * 0.5,
                    preferred_element_type=jnp.float32)
    t = t + c1_ref[...]
    th = 0.5 * t
    u = th * (jnp.tanh(th) + 1.0)
    out_ref[...] = hblk + jnp.dot(u.astype(jnp.bfloat16), U2_ref[...],
                                  preferred_element_type=jnp.float32) + c2_ref[...]


_BN = 1000


def _upd_call(h, aggr_cat, U1h, U1a, c1, U2, c2):
    return pl.pallas_call(
        _upd_body,
        grid=(N // _BN,),
        in_specs=[
            pl.BlockSpec((_BN, H), lambda i: (i, 0)),
            pl.BlockSpec((2, _BN, HH), lambda i: (0, i, 0)),
            pl.BlockSpec((H, H), lambda i: (0, 0)),
            pl.BlockSpec((2, HH, H), lambda i: (0, 0, 0)),
            pl.BlockSpec((1, H), lambda i: (0, 0)),
            pl.BlockSpec((H, H), lambda i: (0, 0)),
            pl.BlockSpec((1, H), lambda i: (0, 0)),
        ],
        out_specs=pl.BlockSpec((_BN, H), lambda i: (i, 0)),
        out_shape=jax.ShapeDtypeStruct((N, H), jnp.float32),
    )(h, aggr_cat, U1h, U1a, c1, U2, c2)


_PERM = np.concatenate([
    np.stack([np.arange(64) + 128 * hh, np.arange(64) + 128 * hh + 64],
             axis=1).reshape(-1)
    for hh in (0, 1)
])


def kernel(h, edge_index, rbf, W1, b1, W2, b2, U1, c1, U2, c2):
    src = edge_index[0]
    dst = edge_index[1]
    gate_cat = _gate_call(rbf, W1, b1.reshape(1, H),
                          W2.astype(jnp.bfloat16), b2.reshape(1, H))
    h2 = h.reshape(2 * N, HH)
    aggr_cat = _sc_call(h2, src, dst, gate_cat)
    return _upd_call(
        h,
        aggr_cat,
        U1[:H].astype(jnp.bfloat16),
        U1[H:].reshape(2, HH, H).astype(jnp.bfloat16),
        c1.reshape(1, H),
        U2.astype(jnp.bfloat16),
        c2.reshape(1, H),
    )


# speedup vs baseline: 1.1406x; 1.0278x over previous
import functools

import numpy as np

import jax
import jax.numpy as jnp
from jax import lax
from jax.experimental import pallas as pl
from jax.experimental.pallas import tpu as pltpu
from jax.experimental.pallas import tpu_sc as plsc

N = 10000
E = 160000
H = 256
HH = H // 2
RBF = 16

NC = 2
NS = 16
K = 64
NCHT = 156
NPAIR = NCHT // 2
EPT = NCHT * K
OUT_ROWS = (N // NS // 8) * 8


def _gate_body(rbf_ref, W1_ref, b1_ref, W2_ref, b2_ref, out_ref):
    t = jnp.dot(rbf_ref[...], W1_ref[...] * 0.5,
                preferred_element_type=jnp.float32)
    t = t + b1_ref[...] * 0.5
    g = t * (jnp.tanh(t) + 1.0)
    z = jnp.dot(g.astype(jnp.bfloat16), W2_ref[...] * 0.5,
                preferred_element_type=jnp.float32) + b2_ref[...] * 0.5
    v = jnp.tanh(z)
    out_ref[0] = v[:, :HH]
    out_ref[1] = v[:, HH:]


_BE = 8000


def _gate_call(rbf, W1, b1, W2, b2):
    return pl.pallas_call(
        _gate_body,
        grid=(E // _BE,),
        in_specs=[
            pl.BlockSpec((_BE, RBF), lambda i: (i, 0)),
            pl.BlockSpec((RBF, H), lambda i: (0, 0)),
            pl.BlockSpec((1, H), lambda i: (0, 0)),
            pl.BlockSpec((H, H), lambda i: (0, 0)),
            pl.BlockSpec((1, H), lambda i: (0, 0)),
        ],
        out_specs=pl.BlockSpec((2, _BE, HH), lambda i: (0, i, 0)),
        out_shape=jax.ShapeDtypeStruct((2, E, HH), jnp.float32),
    )(rbf, W1, b1, W2, b2)


@functools.cache
def _sc_aggr_built():
    mesh = plsc.VectorSubcoreMesh(
        core_axis_name="c", subcore_axis_name="s", num_cores=NC, num_subcores=NS
    )
    return functools.partial(
        pl.kernel,
        out_type=jax.ShapeDtypeStruct((2, N, HH), jnp.float32),
        mesh=mesh,
        scratch_types=(
            [pltpu.VMEM((K,), jnp.int32) for _ in range(4)]
            + [pltpu.VMEM((K,), jnp.int32) for _ in range(4)]
            + [pltpu.VMEM((K, HH), jnp.float32) for _ in range(2)]
            + [pltpu.VMEM((K, HH), jnp.float32) for _ in range(2)]
            + [pltpu.VMEM((K, HH), jnp.float32) for _ in range(2)]
            + [pltpu.VMEM_SHARED((N, HH), jnp.float32)]
            + [pltpu.SemaphoreType.DMA for _ in range(10)]
        ),
    )(_sc_aggr_body)


def _sc_call(h2, src, dst, gcat):
    return _sc_aggr_built()(h2, src, dst, gcat)


def _sc_aggr_body(h2, srcx, dstx, gcat, out,
                  si0, si1, si2, si3, di0, di1, di2, di3,
                  hv0, hv1, gv0, gv1, mv0, mv1, acc,
                  sx0, sx1, sx2, sx3, sh0, sh1, sg0, sg1, ss0, ss1):
    c = lax.axis_index("c")
    s = lax.axis_index("s")
    si = (si0, si1, si2, si3)
    di = (di0, di1, di2, di3)
    sx = (sx0, sx1, sx2, sx3)
    hv = (hv0, hv1)
    gv = (gv0, gv1)
    mv = (mv0, mv1)
    sh = (sh0, sh1)
    sg = (sg0, sg1)
    ss = (ss0, ss1)

    ebase = s * EPT

    def issue_src_idx(j, x):
        e0 = ebase + j * K
        pltpu.make_async_copy(srcx.at[pl.ds(e0, K)], si[x], sx[x]).start()

    def issue_dst_idx(j, x):
        e0 = ebase + j * K
        pltpu.make_async_copy(dstx.at[pl.ds(e0, K)], di[x], sx[x]).start()

    def wait_idx_and_xform(j, x):
        e0 = ebase + j * K
        pltpu.make_async_copy(srcx.at[pl.ds(e0, K)], si[x], sx[x]).wait()
        pltpu.make_async_copy(dstx.at[pl.ds(e0, K)], di[x], sx[x]).wait()
        for q in range(K // 16):
            sl = pl.ds(q * 16, 16)
            si[x][sl] = si[x][sl] * 2 + c

    def issue_loads(j, t, x):
        pltpu.make_async_copy(h2.at[si[x]], hv[t], sh[t]).start()
        pltpu.make_async_copy(
            gcat.at[c, pl.ds(ebase + j * K, K)], gv[t], sg[t]).start()

    def wait_loads(j, t, x):
        pltpu.make_async_copy(h2.at[si[x]], hv[t], sh[t]).wait()
        pltpu.make_async_copy(
            gcat.at[c, pl.ds(ebase + j * K, K)], gv[t], sg[t]).wait()

    def issue_scatter(t, x):
        pltpu.async_copy(mv[t], acc.at[di[x]], ss[t], add=True)

    def wait_scatter(t, x):
        pltpu.make_async_copy(mv[t], acc.at[di[x]], ss[t]).wait()

    def multiply(t):
        @plsc.parallel_loop(0, K, unroll=2)
        def mrow(r):
            for q in range(HH // 16):
                sl = pl.ds(q * 16, 16)
                mv[t][r, sl] = (gv[t][r, sl] + 1.0) * hv[t][r, sl]

    for j in range(4):
        issue_src_idx(j, j)
    issue_dst_idx(0, 0)
    issue_dst_idx(1, 1)
    wait_idx_and_xform(0, 0)
    issue_loads(0, 0, 0)

    def zrow(r2, carry):
        zvec = jnp.zeros((16,), jnp.float32)
        for dr in range(2):
            r = r2 * 2 + dr
            for q in range(HH // 16):
                mv0[r, pl.ds(q * 16, 16)] = zvec
        return carry

    lax.fori_loop(0, K // 2, zrow, 0)
    zbase = s * (N // NS)
    npiece = (N // NS) // K
    for piece in range(npiece):
        pltpu.sync_copy(mv0, acc.at[pl.ds(zbase + piece * K, K)])
    rem = N // NS - npiece * K
    pltpu.sync_copy(mv0.at[pl.ds(0, rem)],
                    acc.at[pl.ds(zbase + npiece * K, rem)])
    plsc.subcore_barrier()

    NQ = NCHT // 4

    def quad(q, carry):
        for u in range(4):
            j = q * 4 + u
            t = u % 2
            if u < 3:
                wait_idx_and_xform(j + 1, u + 1)
                issue_loads(j + 1, 1 - t, u + 1)
            else:
                @pl.when(q < NQ - 1)
                def _next_quad_loads():
                    wait_idx_and_xform(j + 1, 0)
                    issue_loads(j + 1, 1 - t, 0)

            wait_loads(j, t, u)

            if u < 2:
                @pl.when(q >= 1)
                def _wait_prev_scatter():
                    wait_scatter(t, u + 2)

                issue_dst_idx(j + 2, u + 2)
            else:
                wait_scatter(t, u - 2)

                @pl.when(q < NQ - 1)
                def _refill_dst():
                    issue_dst_idx(j + 2, u - 2)

            multiply(t)
            issue_scatter(t, u)

            @pl.when(q < NQ - 1)
            def _refill_src():
                issue_src_idx(j + 4, u)

        return carry

    lax.fori_loop(0, NQ, quad, 0)
    wait_scatter(0, 2)
    wait_scatter(1, 3)

    @pl.when(s < 4)
    def _extra():
        ec = NS * NCHT + s
        e0 = ec * K
        pltpu.sync_copy(srcx.at[pl.ds(e0, K)], si0)
        pltpu.sync_copy(dstx.at[pl.ds(e0, K)], di0)
        for q in range(K // 16):
            sl = pl.ds(q * 16, 16)
            si0[sl] = si0[sl] * 2 + c
        pltpu.async_copy(h2.at[si0], hv0, sh0).wait()
        pltpu.sync_copy(gcat.at[c, pl.ds(e0, K)], gv0)
        multiply(0)
        pltpu.sync_copy(mv0, acc.at[di0], add=True)

    plsc.subcore_barrier()

    r0 = s * OUT_ROWS
    pltpu.sync_copy(acc.at[pl.ds(r0, OUT_ROWS)], out.at[c, pl.ds(r0, OUT_ROWS)])

    @pl.when(s == NS - 1)
    def _tail():
        t0 = NS * OUT_ROWS
        pltpu.sync_copy(acc.at[pl.ds(t0, N - NS * OUT_ROWS)],
                        out.at[c, pl.ds(t0, N - NS * OUT_ROWS)])


def _upd_body(h_ref, a_ref, U1h_ref, U1a_ref, c1_ref, U2_ref, c2_ref, out_ref):
    hblk = h_ref[...]
    t = jnp.dot(hblk.astype(jnp.bfloat16), U1h_ref[...],
                preferred_element_type=jnp.float32)
    t = t + jnp.dot(a_ref[0].astype(jnp.bfloat16), U1a_ref[0] * 0.5,
                    preferred_element_type=jnp.float32)
    t = t + jnp.dot(a_ref[1].astype(jnp.bfloat16), U1a_ref[1] * 0.5,
                    preferred_element_type=jnp.float32)
    t = t + c1_ref[...]
    th = 0.5 * t
    u = th * (jnp.tanh(th) + 1.0)
    out_ref[...] = hblk + jnp.dot(u.astype(jnp.bfloat16), U2_ref[...],
                                  preferred_element_type=jnp.float32) + c2_ref[...]


_BN = 1000


def _upd_call(h, aggr_cat, U1h, U1a, c1, U2, c2):
    return pl.pallas_call(
        _upd_body,
        grid=(N // _BN,),
        in_specs=[
            pl.BlockSpec((_BN, H), lambda i: (i, 0)),
            pl.BlockSpec((2, _BN, HH), lambda i: (0, i, 0)),
            pl.BlockSpec((H, H), lambda i: (0, 0)),
            pl.BlockSpec((2, HH, H), lambda i: (0, 0, 0)),
            pl.BlockSpec((1, H), lambda i: (0, 0)),
            pl.BlockSpec((H, H), lambda i: (0, 0)),
            pl.BlockSpec((1, H), lambda i: (0, 0)),
        ],
        out_specs=pl.BlockSpec((_BN, H), lambda i: (i, 0)),
        out_shape=jax.ShapeDtypeStruct((N, H), jnp.float32),
    )(h, aggr_cat, U1h, U1a, c1, U2, c2)


_PERM = np.concatenate([
    np.stack([np.arange(64) + 128 * hh, np.arange(64) + 128 * hh + 64],
             axis=1).reshape(-1)
    for hh in (0, 1)
])


def kernel(h, edge_index, rbf, W1, b1, W2, b2, U1, c1, U2, c2):
    src = edge_index[0]
    dst = edge_index[1]
    gate_cat = _gate_call(rbf, W1, b1.reshape(1, H),
                          W2.astype(jnp.bfloat16), b2.reshape(1, H))
    h2 = h.reshape(2 * N, HH)
    aggr_cat = _sc_call(h2, src, dst, gate_cat)
    return _upd_call(
        h,
        aggr_cat,
        U1[:H].astype(jnp.bfloat16),
        U1[H:].reshape(2, HH, H).astype(jnp.bfloat16),
        c1.reshape(1, H),
        U2.astype(jnp.bfloat16),
        c2.reshape(1, H),
    )


# speedup vs baseline: 1.1524x; 1.0103x over previous
import functools

import numpy as np

import jax
import jax.numpy as jnp
from jax import lax
from jax.experimental import pallas as pl
from jax.experimental.pallas import tpu as pltpu
from jax.experimental.pallas import tpu_sc as plsc

N = 10000
E = 160000
H = 256
HH = H // 2
RBF = 16

NC = 2
NS = 16
K = 64
NCHT = 156
NPAIR = NCHT // 2
EPT = NCHT * K
OUT_ROWS = (N // NS // 8) * 8


def _gate_body(rbf_ref, W1_ref, b1_ref, W2_ref, b2_ref, out_ref):
    t = jnp.dot(rbf_ref[...], W1_ref[...] * 0.5,
                preferred_element_type=jnp.float32)
    t = t + b1_ref[...] * 0.5
    g = t * (jnp.tanh(t) + 1.0)
    z = jnp.dot(g.astype(jnp.bfloat16), W2_ref[...] * 0.5,
                preferred_element_type=jnp.float32) + b2_ref[...] * 0.5
    v = jnp.tanh(z)
    out_ref[0] = v[:, :HH]
    out_ref[1] = v[:, HH:]


_BE = 16000


def _gate_call(rbf, W1, b1, W2, b2):
    return pl.pallas_call(
        _gate_body,
        grid=(E // _BE,),
        in_specs=[
            pl.BlockSpec((_BE, RBF), lambda i: (i, 0)),
            pl.BlockSpec((RBF, H), lambda i: (0, 0)),
            pl.BlockSpec((1, H), lambda i: (0, 0)),
            pl.BlockSpec((H, H), lambda i: (0, 0)),
            pl.BlockSpec((1, H), lambda i: (0, 0)),
        ],
        out_specs=pl.BlockSpec((2, _BE, HH), lambda i: (0, i, 0)),
        out_shape=jax.ShapeDtypeStruct((2, E, HH), jnp.float32),
    )(rbf, W1, b1, W2, b2)


@functools.cache
def _sc_aggr_built():
    mesh = plsc.VectorSubcoreMesh(
        core_axis_name="c", subcore_axis_name="s", num_cores=NC, num_subcores=NS
    )
    return functools.partial(
        pl.kernel,
        out_type=jax.ShapeDtypeStruct((2, N, HH), jnp.float32),
        mesh=mesh,
        scratch_types=(
            [pltpu.VMEM((K,), jnp.int32) for _ in range(4)]
            + [pltpu.VMEM((K,), jnp.int32) for _ in range(4)]
            + [pltpu.VMEM((K, HH), jnp.float32) for _ in range(2)]
            + [pltpu.VMEM((K, HH), jnp.float32) for _ in range(2)]
            + [pltpu.VMEM((K, HH), jnp.float32) for _ in range(2)]
            + [pltpu.VMEM_SHARED((N, HH), jnp.float32)]
            + [pltpu.SemaphoreType.DMA for _ in range(10)]
        ),
    )(_sc_aggr_body)


def _sc_call(h2, src, dst, gcat):
    return _sc_aggr_built()(h2, src, dst, gcat)


def _sc_aggr_body(h2, srcx, dstx, gcat, out,
                  si0, si1, si2, si3, di0, di1, di2, di3,
                  hv0, hv1, gv0, gv1, mv0, mv1, acc,
                  sx0, sx1, sx2, sx3, sh0, sh1, sg0, sg1, ss0, ss1):
    c = lax.axis_index("c")
    s = lax.axis_index("s")
    si = (si0, si1, si2, si3)
    di = (di0, di1, di2, di3)
    sx = (sx0, sx1, sx2, sx3)
    hv = (hv0, hv1)
    gv = (gv0, gv1)
    mv = (mv0, mv1)
    sh = (sh0, sh1)
    sg = (sg0, sg1)
    ss = (ss0, ss1)

    ebase = s * EPT

    def issue_src_idx(j, x):
        e0 = ebase + j * K
        pltpu.make_async_copy(srcx.at[pl.ds(e0, K)], si[x], sx[x]).start()

    def issue_dst_idx(j, x):
        e0 = ebase + j * K
        pltpu.make_async_copy(dstx.at[pl.ds(e0, K)], di[x], sx[x]).start()

    def wait_idx_and_xform(j, x):
        e0 = ebase + j * K
        pltpu.make_async_copy(srcx.at[pl.ds(e0, K)], si[x], sx[x]).wait()
        pltpu.make_async_copy(dstx.at[pl.ds(e0, K)], di[x], sx[x]).wait()
        for q in range(K // 16):
            sl = pl.ds(q * 16, 16)
            si[x][sl] = si[x][sl] * 2 + c

    def issue_loads(j, t, x):
        pltpu.make_async_copy(h2.at[si[x]], hv[t], sh[t]).start()
        pltpu.make_async_copy(
            gcat.at[c, pl.ds(ebase + j * K, K)], gv[t], sg[t]).start()

    def wait_loads(j, t, x):
        pltpu.make_async_copy(h2.at[si[x]], hv[t], sh[t]).wait()
        pltpu.make_async_copy(
            gcat.at[c, pl.ds(ebase + j * K, K)], gv[t], sg[t]).wait()

    def issue_scatter(t, x):
        pltpu.async_copy(mv[t], acc.at[di[x]], ss[t], add=True)

    def wait_scatter(t, x):
        pltpu.make_async_copy(mv[t], acc.at[di[x]], ss[t]).wait()

    def multiply(t):
        @plsc.parallel_loop(0, K, unroll=2)
        def mrow(r):
            for q in range(HH // 16):
                sl = pl.ds(q * 16, 16)
                mv[t][r, sl] = (gv[t][r, sl] + 1.0) * hv[t][r, sl]

    for j in range(4):
        issue_src_idx(j, j)
    issue_dst_idx(0, 0)
    issue_dst_idx(1, 1)
    wait_idx_and_xform(0, 0)
    issue_loads(0, 0, 0)

    def zrow(r2, carry):
        zvec = jnp.zeros((16,), jnp.float32)
        for dr in range(2):
            r = r2 * 2 + dr
            for q in range(HH // 16):
                mv0[r, pl.ds(q * 16, 16)] = zvec
        return carry

    lax.fori_loop(0, K // 2, zrow, 0)
    zbase = s * (N // NS)
    npiece = (N // NS) // K
    for piece in range(npiece):
        pltpu.sync_copy(mv0, acc.at[pl.ds(zbase + piece * K, K)])
    rem = N // NS - npiece * K
    pltpu.sync_copy(mv0.at[pl.ds(0, rem)],
                    acc.at[pl.ds(zbase + npiece * K, rem)])
    plsc.subcore_barrier()

    NQ = NCHT // 4

    def quad(q, carry):
        for u in range(4):
            j = q * 4 + u
            t = u % 2
            if u < 3:
                wait_idx_and_xform(j + 1, u + 1)
                issue_loads(j + 1, 1 - t, u + 1)
            else:
                @pl.when(q < NQ - 1)
                def _next_quad_loads():
                    wait_idx_and_xform(j + 1, 0)
                    issue_loads(j + 1, 1 - t, 0)

            wait_loads(j, t, u)

            if u < 2:
                @pl.when(q >= 1)
                def _wait_prev_scatter():
                    wait_scatter(t, u + 2)

                issue_dst_idx(j + 2, u + 2)
            else:
                wait_scatter(t, u - 2)

                @pl.when(q < NQ - 1)
                def _refill_dst():
                    issue_dst_idx(j + 2, u - 2)

            multiply(t)
            issue_scatter(t, u)

            @pl.when(q < NQ - 1)
            def _refill_src():
                issue_src_idx(j + 4, u)

        return carry

    lax.fori_loop(0, NQ, quad, 0)
    wait_scatter(0, 2)
    wait_scatter(1, 3)

    @pl.when(s < 4)
    def _extra():
        ec = NS * NCHT + s
        e0 = ec * K
        pltpu.sync_copy(srcx.at[pl.ds(e0, K)], si0)
        pltpu.sync_copy(dstx.at[pl.ds(e0, K)], di0)
        for q in range(K // 16):
            sl = pl.ds(q * 16, 16)
            si0[sl] = si0[sl] * 2 + c
        pltpu.async_copy(h2.at[si0], hv0, sh0).wait()
        pltpu.sync_copy(gcat.at[c, pl.ds(e0, K)], gv0)
        multiply(0)
        pltpu.sync_copy(mv0, acc.at[di0], add=True)

    plsc.subcore_barrier()

    r0 = s * OUT_ROWS
    pltpu.sync_copy(acc.at[pl.ds(r0, OUT_ROWS)], out.at[c, pl.ds(r0, OUT_ROWS)])

    @pl.when(s == NS - 1)
    def _tail():
        t0 = NS * OUT_ROWS
        pltpu.sync_copy(acc.at[pl.ds(t0, N - NS * OUT_ROWS)],
                        out.at[c, pl.ds(t0, N - NS * OUT_ROWS)])


def _upd_body(h_ref, a_ref, U1h_ref, U1a_ref, c1_ref, U2_ref, c2_ref, out_ref):
    hblk = h_ref[...]
    t = jnp.dot(hblk.astype(jnp.bfloat16), U1h_ref[...],
                preferred_element_type=jnp.float32)
    t = t + jnp.dot(a_ref[0].astype(jnp.bfloat16), U1a_ref[0] * 0.5,
                    preferred_element_type=jnp.float32)
    t = t + jnp.dot(a_ref[1].astype(jnp.bfloat16), U1a_ref[1] * 0.5,
                    preferred_element_type=jnp.float32)
    t = t + c1_ref[...]
    th = 0.5 * t
    u = th * (jnp.tanh(th) + 1.0)
    out_ref[...] = hblk + jnp.dot(u.astype(jnp.bfloat16), U2_ref[...],
                                  preferred_element_type=jnp.float32) + c2_ref[...]


_BN = 2000


def _upd_call(h, aggr_cat, U1h, U1a, c1, U2, c2):
    return pl.pallas_call(
        _upd_body,
        grid=(N // _BN,),
        in_specs=[
            pl.BlockSpec((_BN, H), lambda i: (i, 0)),
            pl.BlockSpec((2, _BN, HH), lambda i: (0, i, 0)),
            pl.BlockSpec((H, H), lambda i: (0, 0)),
            pl.BlockSpec((2, HH, H), lambda i: (0, 0, 0)),
            pl.BlockSpec((1, H), lambda i: (0, 0)),
            pl.BlockSpec((H, H), lambda i: (0, 0)),
            pl.BlockSpec((1, H), lambda i: (0, 0)),
        ],
        out_specs=pl.BlockSpec((_BN, H), lambda i: (i, 0)),
        out_shape=jax.ShapeDtypeStruct((N, H), jnp.float32),
    )(h, aggr_cat, U1h, U1a, c1, U2, c2)


_PERM = np.concatenate([
    np.stack([np.arange(64) + 128 * hh, np.arange(64) + 128 * hh + 64],
             axis=1).reshape(-1)
    for hh in (0, 1)
])


def kernel(h, edge_index, rbf, W1, b1, W2, b2, U1, c1, U2, c2):
    src = edge_index[0]
    dst = edge_index[1]
    gate_cat = _gate_call(rbf, W1, b1.reshape(1, H),
                          W2.astype(jnp.bfloat16), b2.reshape(1, H))
    h2 = h.reshape(2 * N, HH)
    aggr_cat = _sc_call(h2, src, dst, gate_cat)
    return _upd_call(
        h,
        aggr_cat,
        U1[:H].astype(jnp.bfloat16),
        U1[H:].reshape(2, HH, H).astype(jnp.bfloat16),
        c1.reshape(1, H),
        U2.astype(jnp.bfloat16),
        c2.reshape(1, H),
    )
